# Initial kernel scaffold; baseline (speedup 1.0000x reference)
import jax
import jax.numpy as jnp
from jax.experimental import pallas as pl


def kernel(x_seq, ei_seq, conv_z_W, conv_z_b, lin_z_W, lin_z_b, conv_r_W, conv_r_b, lin_r_W, lin_r_b, conv_h_W, conv_h_b, lin_h_W, lin_h_b, head_W, head_b):
    raise NotImplementedError("write your pallas kernel here")


# speedup vs baseline: 26.4705x; 26.4705x over previous
import functools

import jax
import jax.numpy as jnp
from jax import lax
from jax.experimental import pallas as pl
from jax.experimental.pallas import tpu as pltpu
from jax.experimental.pallas import tpu_sc as plsc

N = 10000
T = 4
E = 320000
C = 128
HID = 128
NPAD = 10240
CHUNK = 128
NCORE = 2
NSUB = 16


def _sc_mesh():
    return plsc.VectorSubcoreMesh(core_axis_name="c", subcore_axis_name="s")


def _deg_sc(didx, zeros1):
    nchunks = (2 * E) // CHUNK
    base, rem = nchunks // NSUB, nchunks % NSUB
    seg = (2 * NPAD) // NSUB

    @functools.partial(
        pl.kernel,
        out_type=jax.ShapeDtypeStruct((T, NPAD), jnp.float32),
        mesh=_sc_mesh(),
        scratch_types=[
            pltpu.VMEM_SHARED((2 * NPAD,), jnp.float32),
            pltpu.VMEM((CHUNK,), jnp.int32),
            pltpu.VMEM((CHUNK,), jnp.float32),
        ],
    )
    def k(didx_hbm, z_hbm, deg_hbm, acc, idxv, ones):
        cid = lax.axis_index("c")
        sid = lax.axis_index("s")
        for i in range(CHUNK // 16):
            ones[pl.ds(i * 16, 16)] = jnp.full((16,), 1.0, jnp.float32)
        pltpu.sync_copy(z_hbm.at[pl.ds(sid * seg, seg)],
                        acc.at[pl.ds(sid * seg, seg)])
        plsc.subcore_barrier()
        nj = base + jnp.where(sid < rem, 1, 0)

        def body(j, carry):
            ch = sid + NSUB * j
            pltpu.sync_copy(didx_hbm.at[cid, pl.ds(ch * CHUNK, CHUNK)], idxv)
            pltpu.sync_copy(ones, acc.at[idxv], add=True)
            return carry

        lax.fori_loop(0, nj, body, 0)
        plsc.subcore_barrier()
        kk = sid // 8
        off = (sid % 8) * seg
        pltpu.sync_copy(acc.at[pl.ds(sid * seg, seg)],
                        deg_hbm.at[2 * cid + kk, pl.ds(off, seg)])

    return k(didx, zeros1)


def _prep_tc(deg, x_pad):
    BN = 2048
    nb = NPAD // BN

    def body(deg_ref, x_ref, dinv_ref, u_ref):
        dv = lax.rsqrt(deg_ref[...] + 1.0)
        dinv_ref[...] = dv
        u_ref[...] = dv[:, :, None] * x_ref[...]

    return pl.pallas_call(
        body,
        grid=(nb,),
        in_specs=[pl.BlockSpec((T, BN), lambda i: (0, i)),
                  pl.BlockSpec((T, BN, C), lambda i: (0, i, 0))],
        out_specs=[pl.BlockSpec((T, BN), lambda i: (0, i)),
                   pl.BlockSpec((T, BN, C), lambda i: (0, i, 0))],
        out_shape=[jax.ShapeDtypeStruct((T, NPAD), jnp.float32),
                   jax.ShapeDtypeStruct((T, NPAD, C), jnp.float32)],
    )(deg, x_pad)


def _scatter_sc(sidx, didx, u_flat, zeros2):
    nch = E // CHUNK
    base, rem = nch // NSUB, nch % NSUB
    rpt = NPAD // NSUB

    @functools.partial(
        pl.kernel,
        out_type=jax.ShapeDtypeStruct((T, NPAD, C), jnp.float32),
        mesh=_sc_mesh(),
        scratch_types=[
            pltpu.VMEM_SHARED((NPAD, C), jnp.float32),
            pltpu.VMEM((CHUNK,), jnp.int32),
            pltpu.VMEM((CHUNK,), jnp.int32),
            pltpu.VMEM((CHUNK, C), jnp.float32),
        ],
    )
    def k(s_hbm, d_hbm, u_hbm, z_hbm, zout_hbm, acc, siv, div, rows):
        cid = lax.axis_index("c")
        sid = lax.axis_index("s")
        nj = base + jnp.where(sid < rem, 1, 0)
        for kk in range(2):
            pltpu.sync_copy(z_hbm.at[pl.ds(sid * rpt, rpt)],
                            acc.at[pl.ds(sid * rpt, rpt)])
            plsc.subcore_barrier()

            def body(j, carry):
                ch = sid + NSUB * j
                pltpu.sync_copy(s_hbm.at[cid, kk, pl.ds(ch * CHUNK, CHUNK)], siv)
                pltpu.sync_copy(d_hbm.at[cid, kk, pl.ds(ch * CHUNK, CHUNK)], div)
                pltpu.sync_copy(u_hbm.at[siv], rows)
                pltpu.sync_copy(rows, acc.at[div], add=True)
                return carry

            lax.fori_loop(0, nj, body, 0)
            plsc.subcore_barrier()
            pltpu.sync_copy(acc.at[pl.ds(sid * rpt, rpt)],
                            zout_hbm.at[2 * cid + kk, pl.ds(sid * rpt, rpt)])
            plsc.subcore_barrier()

    return k(sidx, didx, u_flat, zeros2)


def _gru_tc(z, u, dinv, czW, czb, lzW, lzb, crW, crb, lrW, lrb,
            chW, chb, lhW, lhb, hW, hb):
    BN = 1024
    nb = NPAD // BN

    def body(z_ref, u_ref, dinv_ref, czW_r, czb_r, lzW_r, lzb_r, crW_r, crb_r,
             lrW_r, lrb_r, chW_r, chb_r, lhW_r, lhb_r, hW_r, hb_r,
             out_ref, acc_ref):
        b = pl.program_id(0)
        Mz = jnp.dot(czW_r[...], lzW_r[0:HID, :])
        Mr = jnp.dot(crW_r[...], lrW_r[0:HID, :])
        Mh = jnp.dot(chW_r[...], lhW_r[0:HID, :])
        cbz = jnp.dot(czb_r[...].reshape(1, HID), lzW_r[0:HID, :]) + lzb_r[...]
        cbr = jnp.dot(crb_r[...].reshape(1, HID), lrW_r[0:HID, :]) + lrb_r[...]
        cbh = jnp.dot(chb_r[...].reshape(1, HID), lhW_r[0:HID, :]) + lhb_r[...]
        Lz2 = lzW_r[HID:, :]
        Lr2 = lrW_r[HID:, :]
        Lh2 = lhW_r[HID:, :]

        h = jnp.zeros((BN, HID), jnp.float32)
        for t in range(T):
            s = dinv_ref[t, :].reshape(BN, 1) * (z_ref[t] + u_ref[t])
            az = jnp.dot(s, Mz) + cbz
            ar = jnp.dot(s, Mr) + cbr
            ah = jnp.dot(s, Mh) + cbh
            Zg = jax.nn.sigmoid(az + jnp.dot(h, Lz2))
            Rg = jax.nn.sigmoid(ar + jnp.dot(h, Lr2))
            Ht = jnp.tanh(ah + jnp.dot(h * Rg, Lh2))
            h = Zg * h + (1.0 - Zg) * Ht

        rowid = lax.broadcasted_iota(jnp.int32, (BN, HID), 0)
        valid = (b * BN + rowid) < N
        part = jnp.sum(jnp.where(valid, h, 0.0), axis=0, keepdims=True)

        @pl.when(b == 0)
        def _():
            acc_ref[...] = jnp.zeros((1, HID), jnp.float32)

        acc_ref[...] += part

        @pl.when(b == nb - 1)
        def _():
            g = acc_ref[...] * (1.0 / N)
            out_ref[...] = (jnp.dot(g, hW_r[...]) + hb_r[...].reshape(1, C)
                            ).reshape(C)

    full = lambda s: pl.BlockSpec(s, lambda i: tuple(0 for _ in s))
    return pl.pallas_call(
        body,
        grid=(nb,),
        in_specs=[pl.BlockSpec((T, BN, C), lambda i: (0, i, 0)),
                  pl.BlockSpec((T, BN, C), lambda i: (0, i, 0)),
                  pl.BlockSpec((T, BN), lambda i: (0, i)),
                  full((HID, HID)), full((HID,)), full((2 * HID, HID)),
                  full((HID,)), full((HID, HID)), full((HID,)),
                  full((2 * HID, HID)), full((HID,)), full((HID, HID)),
                  full((HID,)), full((2 * HID, HID)), full((HID,)),
                  full((HID, C)), full((C,))],
        out_specs=pl.BlockSpec((C,), lambda i: (0,)),
        out_shape=jax.ShapeDtypeStruct((C,), jnp.float32),
        scratch_shapes=[pltpu.VMEM((1, HID), jnp.float32)],
    )(z, u, dinv, czW, czb, lzW, lzb, crW, crb, lrW, lrb,
      chW, chb, lhW, lhb, hW, hb)


def kernel(x_seq, ei_seq, conv_z_W, conv_z_b, lin_z_W, lin_z_b,
           conv_r_W, conv_r_b, lin_r_W, lin_r_b, conv_h_W, conv_h_b,
           lin_h_W, lin_h_b, head_W, head_b):
    ei = ei_seq.astype(jnp.int32)
    src = ei[:, 0, :]
    dst = ei[:, 1, :]
    toff = (jnp.arange(T, dtype=jnp.int32) * NPAD)[:, None]
    koff = ((jnp.arange(T, dtype=jnp.int32) % 2) * NPAD)[:, None]
    sidx = (src + toff).reshape(NCORE, 2, E)
    didx = dst.reshape(NCORE, 2, E)
    didx1 = (dst + koff).reshape(NCORE, 2 * E)
    zeros1 = jnp.zeros((2 * NPAD,), jnp.float32)
    zeros2 = jnp.zeros((NPAD, C), jnp.float32)
    x_pad = jnp.pad(x_seq, ((0, 0), (0, NPAD - N), (0, 0)))

    deg = _deg_sc(didx1, zeros1)
    dinv, u = _prep_tc(deg, x_pad)
    z = _scatter_sc(sidx, didx, u.reshape(T * NPAD, C), zeros2)
    return _gru_tc(z, u, dinv, conv_z_W, conv_z_b, lin_z_W, lin_z_b,
                   conv_r_W, conv_r_b, lin_r_W, lin_r_b, conv_h_W, conv_h_b,
                   lin_h_W, lin_h_b, head_W, head_b)


# speedup vs baseline: 44.0228x; 1.6631x over previous
import functools

import jax
import jax.numpy as jnp
from jax import lax
from jax.experimental import pallas as pl
from jax.experimental.pallas import tpu as pltpu
from jax.experimental.pallas import tpu_sc as plsc

N = 10000
T = 4
E = 320000
C = 128
HID = 128
NPAD = 10240
CHUNK = 128
NCORE = 2
NSUB = 16

NCH1 = -(-2 * E // (CHUNK * NSUB)) * NSUB
NCH3 = -(-E // (CHUNK * NSUB)) * NSUB


def _sc_mesh():
    return plsc.VectorSubcoreMesh(core_axis_name="c", subcore_axis_name="s")


def _deg_sc(didx, zeros1):
    per = NCH1 // NSUB
    seg = (2 * NPAD) // NSUB

    @functools.partial(
        pl.kernel,
        out_type=jax.ShapeDtypeStruct((T, NPAD), jnp.float32),
        mesh=_sc_mesh(),
        scratch_types=[
            pltpu.VMEM_SHARED((2 * NPAD,), jnp.float32),
            pltpu.VMEM((CHUNK,), jnp.int32),
            pltpu.VMEM((CHUNK,), jnp.int32),
            pltpu.VMEM((CHUNK,), jnp.float32),
        ] + [pltpu.SemaphoreType.DMA] * 2,
    )
    def k(didx_hbm, z_hbm, deg_hbm, acc, ib0, ib1, ones, is0, is1):
        cid = lax.axis_index("c")
        sid = lax.axis_index("s")
        for i in range(CHUNK // 16):
            ones[pl.ds(i * 16, 16)] = jnp.full((16,), 1.0, jnp.float32)
        pltpu.sync_copy(z_hbm.at[pl.ds(sid * seg, seg)],
                        acc.at[pl.ds(sid * seg, seg)])
        plsc.subcore_barrier()
        cb = sid * per

        def iload(c, buf, sem):
            return pltpu.async_copy(
                didx_hbm.at[cid, pl.ds((cb + c) * CHUNK, CHUNK)], buf, sem)

        def iwait(c, buf, sem):
            pltpu.make_async_copy(
                didx_hbm.at[cid, pl.ds((cb + c) * CHUNK, CHUNK)],
                buf, sem).wait()

        pltpu.sync_copy(didx_hbm.at[cid, pl.ds(cb * CHUNK, CHUNK)], ib0)

        def body(j, carry):
            c = 2 * j
            iload(c + 1, ib1, is1)
            pltpu.sync_copy(ones, acc.at[ib0], add=True)
            iload(c + 2, ib0, is0)
            iwait(c + 1, ib1, is1)
            pltpu.sync_copy(ones, acc.at[ib1], add=True)
            iwait(c + 2, ib0, is0)
            return carry

        lax.fori_loop(0, per // 2 - 1, body, 0)
        iload(per - 1, ib1, is1)
        pltpu.sync_copy(ones, acc.at[ib0], add=True)
        iwait(per - 1, ib1, is1)
        pltpu.sync_copy(ones, acc.at[ib1], add=True)
        plsc.subcore_barrier()
        kk = sid // 8
        off = (sid % 8) * seg
        pltpu.sync_copy(acc.at[pl.ds(sid * seg, seg)],
                        deg_hbm.at[2 * cid + kk, pl.ds(off, seg)])

    return k(didx, zeros1)


def _prep_tc(deg, x_pad):
    BN = 2048
    nb = NPAD // BN

    def body(deg_ref, x_ref, dinv_ref, u_ref):
        dv = lax.rsqrt(deg_ref[...] + 1.0)
        dinv_ref[...] = dv
        u_ref[...] = dv[:, :, None] * x_ref[...]

    return pl.pallas_call(
        body,
        grid=(nb,),
        in_specs=[pl.BlockSpec((T, BN), lambda i: (0, i)),
                  pl.BlockSpec((T, BN, C), lambda i: (0, i, 0))],
        out_specs=[pl.BlockSpec((T, BN), lambda i: (0, i)),
                   pl.BlockSpec((T, BN, C), lambda i: (0, i, 0))],
        out_shape=[jax.ShapeDtypeStruct((T, NPAD), jnp.float32),
                   jax.ShapeDtypeStruct((T, NPAD, C), jnp.float32)],
    )(deg, x_pad)


def _scatter_sc(sidx, didx, u_flat, zeros2):
    per = NCH3 // NSUB
    nq = per // 4
    rpt = NPAD // NSUB

    @functools.partial(
        pl.kernel,
        out_type=jax.ShapeDtypeStruct((T, NPAD, C), jnp.float32),
        mesh=_sc_mesh(),
        scratch_types=[
            pltpu.VMEM_SHARED((NPAD, C), jnp.float32),
        ] + [pltpu.VMEM((CHUNK,), jnp.int32)] * 8 + [
            pltpu.VMEM((2, CHUNK, C), jnp.float32),
        ] + [pltpu.SemaphoreType.DMA] * 6,
    )
    def k(s_hbm, d_hbm, u_hbm, z_hbm, zout_hbm, acc,
          sv0, sv1, sv2, sv3, dv0, dv1, dv2, dv3, rows,
          g0, g1, i0, i1, i2, i3):
        siv = (sv0, sv1, sv2, sv3)
        div = (dv0, dv1, dv2, dv3)
        gsem = (g0, g1)
        isem = (i0, i1, i2, i3)
        cid = lax.axis_index("c")
        sid = lax.axis_index("s")
        cb = sid * per

        for kk in range(2):
            def idx_start(c, slot):
                pltpu.async_copy(
                    s_hbm.at[cid, kk, pl.ds((cb + c) * CHUNK, CHUNK)],
                    siv[slot], isem[slot])
                pltpu.async_copy(
                    d_hbm.at[cid, kk, pl.ds((cb + c) * CHUNK, CHUNK)],
                    div[slot], isem[slot])

            def idx_wait(c, slot):
                pltpu.make_async_copy(
                    s_hbm.at[cid, kk, pl.ds((cb + c) * CHUNK, CHUNK)],
                    siv[slot], isem[slot]).wait()
                pltpu.make_async_copy(
                    d_hbm.at[cid, kk, pl.ds((cb + c) * CHUNK, CHUNK)],
                    div[slot], isem[slot]).wait()

            pltpu.sync_copy(z_hbm.at[pl.ds(sid * rpt, rpt)],
                            acc.at[pl.ds(sid * rpt, rpt)])
            plsc.subcore_barrier()
            for b in range(4):
                idx_start(b, b)
            for b in range(2):
                idx_wait(b, b)
                pltpu.async_copy(u_hbm.at[siv[b]], rows.at[b], gsem[b])

            def body(q, carry):
                for i in range(4):
                    c = 4 * q + i
                    pltpu.make_async_copy(u_hbm.at[siv[i]], rows.at[i % 2],
                                          gsem[i % 2]).wait()
                    pltpu.sync_copy(rows.at[i % 2], acc.at[div[i]],
                                    add=True)
                    idx_start(c + 4, i)
                    idx_wait(c + 2, (i + 2) % 4)
                    pltpu.async_copy(u_hbm.at[siv[(i + 2) % 4]],
                                     rows.at[i % 2], gsem[i % 2])
                return carry

            lax.fori_loop(0, nq - 1, body, 0)
            for i in range(4):
                c = per - 4 + i
                pltpu.make_async_copy(u_hbm.at[siv[i]], rows.at[i % 2],
                                      gsem[i % 2]).wait()
                pltpu.sync_copy(rows.at[i % 2], acc.at[div[i]], add=True)
                if i < 2:
                    idx_wait(c + 2, (i + 2) % 4)
                    pltpu.async_copy(u_hbm.at[siv[(i + 2) % 4]],
                                     rows.at[i % 2], gsem[i % 2])
            plsc.subcore_barrier()
            pltpu.sync_copy(acc.at[pl.ds(sid * rpt, rpt)],
                            zout_hbm.at[2 * cid + kk, pl.ds(sid * rpt, rpt)])
            plsc.subcore_barrier()

    return k(sidx, didx, u_flat, zeros2)


def _gru_tc(z, u, dinv, czW, czb, lzW, lzb, crW, crb, lrW, lrb,
            chW, chb, lhW, lhb, hW, hb):
    BN = 1024
    nb = NPAD // BN

    def body(z_ref, u_ref, dinv_ref, czW_r, czb_r, lzW_r, lzb_r, crW_r, crb_r,
             lrW_r, lrb_r, chW_r, chb_r, lhW_r, lhb_r, hW_r, hb_r,
             out_ref, acc_ref):
        b = pl.program_id(0)
        Mz = jnp.dot(czW_r[...], lzW_r[0:HID, :])
        Mr = jnp.dot(crW_r[...], lrW_r[0:HID, :])
        Mh = jnp.dot(chW_r[...], lhW_r[0:HID, :])
        cbz = jnp.dot(czb_r[...].reshape(1, HID), lzW_r[0:HID, :]) + lzb_r[...]
        cbr = jnp.dot(crb_r[...].reshape(1, HID), lrW_r[0:HID, :]) + lrb_r[...]
        cbh = jnp.dot(chb_r[...].reshape(1, HID), lhW_r[0:HID, :]) + lhb_r[...]
        Lz2 = lzW_r[HID:, :]
        Lr2 = lrW_r[HID:, :]
        Lh2 = lhW_r[HID:, :]

        h = jnp.zeros((BN, HID), jnp.float32)
        for t in range(T):
            s = dinv_ref[t, :].reshape(BN, 1) * (z_ref[t] + u_ref[t])
            az = jnp.dot(s, Mz) + cbz
            ar = jnp.dot(s, Mr) + cbr
            ah = jnp.dot(s, Mh) + cbh
            Zg = jax.nn.sigmoid(az + jnp.dot(h, Lz2))
            Rg = jax.nn.sigmoid(ar + jnp.dot(h, Lr2))
            Ht = jnp.tanh(ah + jnp.dot(h * Rg, Lh2))
            h = Zg * h + (1.0 - Zg) * Ht

        rowid = lax.broadcasted_iota(jnp.int32, (BN, HID), 0)
        valid = (b * BN + rowid) < N
        part = jnp.sum(jnp.where(valid, h, 0.0), axis=0, keepdims=True)

        @pl.when(b == 0)
        def _():
            acc_ref[...] = jnp.zeros((1, HID), jnp.float32)

        acc_ref[...] += part

        @pl.when(b == nb - 1)
        def _():
            g = acc_ref[...] * (1.0 / N)
            out_ref[...] = (jnp.dot(g, hW_r[...]) + hb_r[...].reshape(1, C)
                            ).reshape(C)

    full = lambda s: pl.BlockSpec(s, lambda i: tuple(0 for _ in s))
    return pl.pallas_call(
        body,
        grid=(nb,),
        in_specs=[pl.BlockSpec((T, BN, C), lambda i: (0, i, 0)),
                  pl.BlockSpec((T, BN, C), lambda i: (0, i, 0)),
                  pl.BlockSpec((T, BN), lambda i: (0, i)),
                  full((HID, HID)), full((HID,)), full((2 * HID, HID)),
                  full((HID,)), full((HID, HID)), full((HID,)),
                  full((2 * HID, HID)), full((HID,)), full((HID, HID)),
                  full((HID,)), full((2 * HID, HID)), full((HID,)),
                  full((HID, C)), full((C,))],
        out_specs=pl.BlockSpec((C,), lambda i: (0,)),
        out_shape=jax.ShapeDtypeStruct((C,), jnp.float32),
        scratch_shapes=[pltpu.VMEM((1, HID), jnp.float32)],
    )(z, u, dinv, czW, czb, lzW, lzb, crW, crb, lrW, lrb,
      chW, chb, lhW, lhb, hW, hb)


def kernel(x_seq, ei_seq, conv_z_W, conv_z_b, lin_z_W, lin_z_b,
           conv_r_W, conv_r_b, lin_r_W, lin_r_b, conv_h_W, conv_h_b,
           lin_h_W, lin_h_b, head_W, head_b):
    ei = ei_seq.astype(jnp.int32)
    src = ei[:, 0, :]
    dst = ei[:, 1, :]
    toff = (jnp.arange(T, dtype=jnp.int32) * NPAD)[:, None]
    koff = ((jnp.arange(T, dtype=jnp.int32) % 2) * NPAD)[:, None]

    npad3 = NCH3 * CHUNK - E
    s_pad = (jnp.arange(T, dtype=jnp.int32) * NPAD + (NPAD - 1))[:, None]
    s_pad = jnp.broadcast_to(s_pad, (T, npad3))
    d_pad = jnp.full((T, npad3), NPAD - 1, jnp.int32)
    sidx = jnp.concatenate([src + toff, s_pad], axis=1).reshape(
        NCORE, 2, NCH3 * CHUNK)
    didx = jnp.concatenate([dst, d_pad], axis=1).reshape(
        NCORE, 2, NCH3 * CHUNK)

    npad1 = NCH1 * CHUNK - 2 * E
    dk = (dst + koff).reshape(NCORE, 2 * E)
    d1_pad = jnp.full((NCORE, npad1), NPAD - 1, jnp.int32)
    didx1 = jnp.concatenate([dk, d1_pad], axis=1)

    zeros1 = jnp.zeros((2 * NPAD,), jnp.float32)
    zeros2 = jnp.zeros((NPAD, C), jnp.float32)
    x_pad = jnp.pad(x_seq, ((0, 0), (0, NPAD - N), (0, 0)))

    deg = _deg_sc(didx1, zeros1)
    dinv, u = _prep_tc(deg, x_pad)
    z = _scatter_sc(sidx, didx, u.reshape(T * NPAD, C), zeros2)
    return _gru_tc(z, u, dinv, conv_z_W, conv_z_b, lin_z_W, lin_z_b,
                   conv_r_W, conv_r_b, lin_r_W, lin_r_b, conv_h_W, conv_h_b,
                   lin_h_W, lin_h_b, head_W, head_b)
